# baseline (speedup 1.0000x reference)
import functools

import jax
import jax.numpy as jnp
from jax import lax
from jax.experimental import pallas as pl
from jax.experimental.pallas import tpu as pltpu
from jax.experimental.pallas import tpu_sc as plsc

N_TABLE = 100000
NB = 10000
S = 32
D = 128

NW = 32
PW = 320
CH = 128
NSTRM = PW * S + 512
TRASH = 320
AGG_ROWS = 336
OUT_CH = 64
NBUF = 4
PSH = 17
PMASK = (1 << PSH) - 1


def _sc_gather_agg(feats, samp_flat, nodes_pad, lens_exp, zrows):
    mesh = plsc.VectorSubcoreMesh(core_axis_name="c", subcore_axis_name="s")

    @functools.partial(
        pl.kernel,
        out_type=(
            jax.ShapeDtypeStruct((NB, D), jnp.float32),
            jax.ShapeDtypeStruct((NB, D), jnp.float32),
        ),
        mesh=mesh,
        scratch_types=[
            pltpu.VMEM((NSTRM,), jnp.int32),
            pltpu.VMEM((PW * S,), jnp.int32),
            pltpu.VMEM((NBUF, CH), jnp.int32),
            pltpu.VMEM((NBUF, CH), jnp.int32),
            pltpu.VMEM((PW,), jnp.int32),
            pltpu.VMEM((NBUF, CH, D), jnp.float32),
            pltpu.VMEM_SHARED((16 * AGG_ROWS, D), jnp.float32),
            [pltpu.SemaphoreType.DMA] * NBUF,
            [pltpu.SemaphoreType.DMA] * NBUF,
            pltpu.SemaphoreType.DMA,
        ],
        compiler_params=pltpu.CompilerParams(needs_layout_passes=False),
    )
    def k(feats_h, samp_h, nodes_h, lensx_h, z_h, agg_out, self_out,
          strm_v, lensb_v, gidx_v, sseg_v, nodes_v, bufs, agg_sh,
          gsem, ssem, osem):
        cid = lax.axis_index("c")
        sid = lax.axis_index("s")
        wid = sid * 2 + cid
        bn = jnp.minimum(wid * PW, NB - PW)
        base = sid * AGG_ROWS

        zcp = pltpu.async_copy(z_h, agg_sh.at[pl.ds(base, AGG_ROWS)], osem)

        cp1 = pltpu.async_copy(samp_h.at[pl.ds(bn * S, PW * S)],
                               strm_v.at[pl.ds(0, PW * S)], gsem[0])
        cp2 = pltpu.async_copy(lensx_h.at[pl.ds(bn * S, PW * S)],
                               lensb_v, gsem[1])
        cp3 = pltpu.async_copy(nodes_h.at[pl.ds(bn, PW)], nodes_v,
                               gsem[2])
        cp1.wait()
        cp2.wait()
        cp3.wait()

        DUMP = NSTRM - 16

        def cbody(v, off_v):
            p0 = v * 16
            samp16 = strm_v[pl.ds(p0, 16)]
            lens16 = lensb_v[pl.ds(p0, 16)]
            p = p0 + lax.iota(jnp.int32, 16)
            i = lax.shift_right_logical(p, 5)
            j = jnp.bitwise_and(p, S - 1)
            m = j < lens16
            packed = jnp.bitwise_or(samp16, jnp.left_shift(i, PSH))
            excl = plsc.cumsum(m.astype(jnp.int32)) - 1
            pos = jnp.where(m, off_v + excl, DUMP)
            plsc.store_scatter(strm_v, [pos], packed)
            return off_v + plsc.all_reduce_population_count(m)

        off_v = pl.loop(0, PW * S // 16,
                        init_carry=jnp.zeros((16,), jnp.int32))(cbody)
        off = jnp.max(off_v)

        padw = jnp.full((16,), TRASH << PSH, jnp.int32)
        for kk in range(512 // 16):
            strm_v[pl.ds(off + kk * 16, 16)] = padw
        nch = lax.shift_left(lax.shift_right_logical(off + 511, 9), 2)

        zcp.wait()

        def prep(c, b):
            for kk in range(CH // 16):
                w = strm_v[pl.ds(c * CH + kk * 16, 16)]
                gidx_v[b, pl.ds(kk * 16, 16)] = jnp.bitwise_and(w, PMASK)
                sseg_v[b, pl.ds(kk * 16, 16)] = (
                    lax.shift_right_logical(w, PSH) + base)

        def start_g(b):
            pltpu.async_copy(feats_h.at[gidx_v.at[b]], bufs.at[b], gsem[b])

        def wait_g(b):
            pltpu.make_async_copy(feats_h.at[gidx_v.at[b]], bufs.at[b],
                                  gsem[b]).wait()

        def wait_s(b):
            pltpu.make_async_copy(z_h.at[pl.ds(0, CH)], bufs.at[b],
                                  ssem[b]).wait()

        @pl.when(nch > 0)
        def _():
            prep(0, 0)
            start_g(0)
            prep(1, 1)
            start_g(1)

        @pl.loop(0, lax.shift_right_logical(nch, 2))
        def _main(t):
            for b in range(NBUF):
                c = t * NBUF + b
                wait_g(b)
                pltpu.async_copy(bufs.at[b], agg_sh.at[sseg_v.at[b]],
                                 ssem[b], add=True)
                b2 = (b + 2) % NBUF

                @pl.when(c >= 2)
                def _():
                    wait_s(b2)

                @pl.when(c + 2 < nch)
                def _():
                    prep(c + 2, b2)
                    start_g(b2)

        @pl.when(nch > 0)
        def _():
            wait_s(2)
            wait_s(3)

        def self_g(t, b, n):
            pltpu.async_copy(feats_h.at[nodes_v.at[pl.ds(t * CH, n)]],
                             bufs.at[b].at[pl.ds(0, n)], gsem[b])
            pltpu.make_async_copy(feats_h.at[nodes_v.at[pl.ds(t * CH, n)]],
                                  bufs.at[b].at[pl.ds(0, n)], gsem[b]).wait()
            pltpu.async_copy(bufs.at[b].at[pl.ds(0, n)],
                             self_out.at[pl.ds(bn + t * CH, n)], osem)

        self_g(0, 0, CH)
        self_g(1, 1, CH)
        self_g(2, 2, PW - 2 * CH)

        for t in range(PW // OUT_CH):
            pltpu.async_copy(
                agg_sh.at[pl.ds(base + t * OUT_CH, OUT_CH)],
                agg_out.at[pl.ds(bn + t * OUT_CH, OUT_CH)], osem)
        for t in range(2):
            pltpu.make_async_copy(
                z_h.at[pl.ds(0, CH)],
                agg_out.at[pl.ds(bn, CH)], osem).wait()
        pltpu.make_async_copy(
            z_h.at[pl.ds(0, PW - 2 * CH)],
            agg_out.at[pl.ds(bn, PW - 2 * CH)], osem).wait()
        for t in range(PW // OUT_CH):
            pltpu.make_async_copy(
                z_h.at[pl.ds(0, OUT_CH)],
                agg_out.at[pl.ds(bn, OUT_CH)], osem).wait()

    return k(feats, samp_flat, nodes_pad, lens_exp, zrows)


def _tc_dense(self_e, agg_sum, lensf, w1a, w1b):
    BLK = 400

    def body(self_ref, agg_ref, lens_ref, wa_ref, wb_ref, out_ref):
        recip = 1.0 / jnp.maximum(lens_ref[...], 1.0)
        h_self = lax.dot_general(self_ref[...], wa_ref[...],
                                 (((1,), (1,)), ((), ())),
                                 preferred_element_type=jnp.float32)
        h_agg = lax.dot_general(agg_ref[...], wb_ref[...],
                                (((1,), (1,)), ((), ())),
                                preferred_element_type=jnp.float32)
        out_ref[...] = jnp.maximum(h_self + recip * h_agg, 0.0)

    return pl.pallas_call(
        body,
        grid=(NB // BLK,),
        in_specs=[
            pl.BlockSpec((BLK, D), lambda i: (i, 0)),
            pl.BlockSpec((BLK, D), lambda i: (i, 0)),
            pl.BlockSpec((BLK, 1), lambda i: (i, 0)),
            pl.BlockSpec((D, D), lambda i: (0, 0)),
            pl.BlockSpec((D, D), lambda i: (0, 0)),
        ],
        out_specs=pl.BlockSpec((BLK, D), lambda i: (i, 0)),
        out_shape=jax.ShapeDtypeStruct((NB, D), jnp.float32),
    )(self_e, agg_sum, lensf, w1a, w1b)


def kernel(nodes, samp_neighs, val_lens, feats_data, W0, W1):
    del W0
    nodes_i = nodes.astype(jnp.int32)
    samp_i = samp_neighs.astype(jnp.int32).reshape(-1)
    lens_i = val_lens.astype(jnp.int32)
    zrows = jnp.zeros((AGG_ROWS, D), jnp.float32)
    lens_exp = jnp.repeat(lens_i, S)
    agg_sum, self_e = _sc_gather_agg(feats_data, samp_i, nodes_i,
                                     lens_exp, zrows)
    lensf = lens_i.astype(jnp.float32).reshape(NB, 1)
    return _tc_dense(self_e, agg_sum, lensf, W1[:, :D], W1[:, D:])


# speedup vs baseline: 1.0073x; 1.0073x over previous
import functools

import jax
import jax.numpy as jnp
from jax import lax
from jax.experimental import pallas as pl
from jax.experimental.pallas import tpu as pltpu
from jax.experimental.pallas import tpu_sc as plsc

N_TABLE = 100000
NB = 10000
S = 32
D = 128

NW = 32
PW = 320
CH = 128
NSTRM = PW * S + 512
TRASH = 320
AGG_ROWS = 336
OUT_CH = 64
NBUF = 4
PSH = 17
PMASK = (1 << PSH) - 1


def _sc_gather_agg(feats, samp_flat, nodes_pad, lens_exp, zrows):
    mesh = plsc.VectorSubcoreMesh(core_axis_name="c", subcore_axis_name="s")

    @functools.partial(
        pl.kernel,
        out_type=(
            jax.ShapeDtypeStruct((NB, D), jnp.float32),
            jax.ShapeDtypeStruct((NB, D), jnp.float32),
        ),
        mesh=mesh,
        scratch_types=[
            pltpu.VMEM((NSTRM,), jnp.int32),
            pltpu.VMEM((PW * S,), jnp.int32),
            pltpu.VMEM((NBUF, CH), jnp.int32),
            pltpu.VMEM((NBUF, CH), jnp.int32),
            pltpu.VMEM((PW,), jnp.int32),
            pltpu.VMEM((NBUF, CH, D), jnp.float32),
            pltpu.VMEM_SHARED((16 * AGG_ROWS, D), jnp.float32),
            [pltpu.SemaphoreType.DMA] * NBUF,
            [pltpu.SemaphoreType.DMA] * NBUF,
            pltpu.SemaphoreType.DMA,
        ],
        compiler_params=pltpu.CompilerParams(needs_layout_passes=False),
    )
    def k(feats_h, samp_h, nodes_h, lensx_h, z_h, agg_out, self_out,
          strm_v, lensb_v, gidx_v, sseg_v, nodes_v, bufs, agg_sh,
          gsem, ssem, osem):
        cid = lax.axis_index("c")
        sid = lax.axis_index("s")
        wid = sid * 2 + cid
        bn = jnp.minimum(wid * PW, NB - PW)
        base = sid * AGG_ROWS

        zcp = pltpu.async_copy(z_h, agg_sh.at[pl.ds(base, AGG_ROWS)], osem)

        cp1 = pltpu.async_copy(samp_h.at[pl.ds(bn * S, PW * S)],
                               strm_v.at[pl.ds(0, PW * S)], gsem[0])
        cp2 = pltpu.async_copy(lensx_h.at[pl.ds(bn * S, PW * S)],
                               lensb_v, gsem[1])
        cp3 = pltpu.async_copy(nodes_h.at[pl.ds(bn, PW)], nodes_v,
                               gsem[2])
        cp1.wait()
        cp2.wait()
        cp3.wait()

        def cbody(v, off):
            p0 = v * 16
            samp16 = strm_v[pl.ds(p0, 16)]
            lens16 = lensb_v[pl.ds(p0, 16)]
            p = p0 + lax.iota(jnp.int32, 16)
            i = lax.shift_right_logical(p, 5)
            j = jnp.bitwise_and(p, S - 1)
            m = j < lens16
            packed = jnp.bitwise_or(samp16, jnp.left_shift(i, PSH))
            plsc.store_compressed(strm_v.at[pl.ds(off, 16)], packed, mask=m)
            cnt = jnp.max(plsc.all_reduce_population_count(m))
            return off + cnt

        off = pl.loop(0, PW * S // 16, init_carry=jnp.int32(0))(cbody)

        padw = jnp.full((16,), TRASH << PSH, jnp.int32)
        for kk in range(512 // 16):
            strm_v[pl.ds(off + kk * 16, 16)] = padw
        nch = lax.shift_left(lax.shift_right_logical(off + 511, 9), 2)

        zcp.wait()

        def prep(c, b):
            for kk in range(CH // 16):
                w = strm_v[pl.ds(c * CH + kk * 16, 16)]
                gidx_v[b, pl.ds(kk * 16, 16)] = jnp.bitwise_and(w, PMASK)
                sseg_v[b, pl.ds(kk * 16, 16)] = (
                    lax.shift_right_logical(w, PSH) + base)

        def start_g(b):
            pltpu.async_copy(feats_h.at[gidx_v.at[b]], bufs.at[b], gsem[b])

        def wait_g(b):
            pltpu.make_async_copy(feats_h.at[gidx_v.at[b]], bufs.at[b],
                                  gsem[b]).wait()

        def wait_s(b):
            pltpu.make_async_copy(z_h.at[pl.ds(0, CH)], bufs.at[b],
                                  ssem[b]).wait()

        @pl.when(nch > 0)
        def _():
            prep(0, 0)
            start_g(0)
            prep(1, 1)
            start_g(1)

        @pl.loop(0, lax.shift_right_logical(nch, 2))
        def _main(t):
            for b in range(NBUF):
                c = t * NBUF + b
                wait_g(b)
                pltpu.async_copy(bufs.at[b], agg_sh.at[sseg_v.at[b]],
                                 ssem[b], add=True)
                b2 = (b + 2) % NBUF

                @pl.when(c >= 2)
                def _():
                    wait_s(b2)

                @pl.when(c + 2 < nch)
                def _():
                    prep(c + 2, b2)
                    start_g(b2)

        @pl.when(nch > 0)
        def _():
            wait_s(2)
            wait_s(3)

        def self_g(t, b, n):
            pltpu.async_copy(feats_h.at[nodes_v.at[pl.ds(t * CH, n)]],
                             bufs.at[b].at[pl.ds(0, n)], gsem[b])
            pltpu.make_async_copy(feats_h.at[nodes_v.at[pl.ds(t * CH, n)]],
                                  bufs.at[b].at[pl.ds(0, n)], gsem[b]).wait()
            pltpu.async_copy(bufs.at[b].at[pl.ds(0, n)],
                             self_out.at[pl.ds(bn + t * CH, n)], osem)

        self_g(0, 0, CH)
        self_g(1, 1, CH)
        self_g(2, 2, PW - 2 * CH)

        for t in range(PW // OUT_CH):
            pltpu.async_copy(
                agg_sh.at[pl.ds(base + t * OUT_CH, OUT_CH)],
                agg_out.at[pl.ds(bn + t * OUT_CH, OUT_CH)], osem)
        for t in range(2):
            pltpu.make_async_copy(
                z_h.at[pl.ds(0, CH)],
                agg_out.at[pl.ds(bn, CH)], osem).wait()
        pltpu.make_async_copy(
            z_h.at[pl.ds(0, PW - 2 * CH)],
            agg_out.at[pl.ds(bn, PW - 2 * CH)], osem).wait()
        for t in range(PW // OUT_CH):
            pltpu.make_async_copy(
                z_h.at[pl.ds(0, OUT_CH)],
                agg_out.at[pl.ds(bn, OUT_CH)], osem).wait()

    return k(feats, samp_flat, nodes_pad, lens_exp, zrows)


def _tc_dense(self_e, agg_sum, lensf, w1a, w1b):
    BLK = 400

    def body(self_ref, agg_ref, lens_ref, wa_ref, wb_ref, out_ref):
        recip = 1.0 / jnp.maximum(lens_ref[...], 1.0)
        h_self = lax.dot_general(self_ref[...], wa_ref[...],
                                 (((1,), (1,)), ((), ())),
                                 preferred_element_type=jnp.float32)
        h_agg = lax.dot_general(agg_ref[...], wb_ref[...],
                                (((1,), (1,)), ((), ())),
                                preferred_element_type=jnp.float32)
        out_ref[...] = jnp.maximum(h_self + recip * h_agg, 0.0)

    return pl.pallas_call(
        body,
        grid=(NB // BLK,),
        in_specs=[
            pl.BlockSpec((BLK, D), lambda i: (i, 0)),
            pl.BlockSpec((BLK, D), lambda i: (i, 0)),
            pl.BlockSpec((BLK, 1), lambda i: (i, 0)),
            pl.BlockSpec((D, D), lambda i: (0, 0)),
            pl.BlockSpec((D, D), lambda i: (0, 0)),
        ],
        out_specs=pl.BlockSpec((BLK, D), lambda i: (i, 0)),
        out_shape=jax.ShapeDtypeStruct((NB, D), jnp.float32),
    )(self_e, agg_sum, lensf, w1a, w1b)


def kernel(nodes, samp_neighs, val_lens, feats_data, W0, W1):
    del W0
    nodes_i = nodes.astype(jnp.int32)
    samp_i = samp_neighs.astype(jnp.int32).reshape(-1)
    lens_i = val_lens.astype(jnp.int32)
    zrows = jnp.zeros((AGG_ROWS, D), jnp.float32)
    lens_exp = jnp.repeat(lens_i, S)
    agg_sum, self_e = _sc_gather_agg(feats_data, samp_i, nodes_i,
                                     lens_exp, zrows)
    lensf = lens_i.astype(jnp.float32).reshape(NB, 1)
    return _tc_dense(self_e, agg_sum, lensf, W1[:, :D], W1[:, D:])
